# baseline (speedup 1.0000x reference)
import functools

import jax
import jax.numpy as jnp
from jax import lax
from jax.experimental import pallas as pl
from jax.experimental.pallas import tpu as pltpu
from jax.experimental.pallas import tpu_sc as plsc

N = 10000
E = 320000
H = 128
NC = 2
NS = 16
NW = NC * NS
CH = 64
NCHUNK = 160
E_PAD = NW * NCHUNK * CH
DEPTH = 2
SEG = 16
NSEG = NCHUNK // SEG

N_ACC = 10112
ZROWS = N_ACC // NS


def _sc_aggregate(xp, srcq, pidx, dstr, zeros):
    mesh = plsc.VectorSubcoreMesh(core_axis_name="c", subcore_axis_name="s")

    @functools.partial(
        pl.kernel,
        out_type=jax.ShapeDtypeStruct((NC, N_ACC, H), jnp.float32),
        mesh=mesh,
        scratch_types=[
            pltpu.VMEM((SEG, CH), jnp.int32),
            pltpu.VMEM((SEG, CH), jnp.int32),
            pltpu.VMEM((SEG, CH), jnp.int32),
            [pltpu.VMEM((CH, 2 * H), jnp.float32) for _ in range(DEPTH)],
            pltpu.VMEM((CH, H), jnp.float32),
            pltpu.VMEM_SHARED((N_ACC, H), jnp.float32),
            [pltpu.SemaphoreType.DMA for _ in range(DEPTH)],
        ],
    )
    def agg_kernel(xp_hbm, srcq_hbm, pidx_hbm, dstr_hbm, zeros_hbm, out_hbm,
                   srcq_iv, pidx_iv, dstr_iv, bufs, cbuf, acc_sh, gsems):
        c = lax.axis_index("c")
        s = lax.axis_index("s")
        wid = c * NS + s

        pltpu.sync_copy(zeros_hbm, acc_sh.at[pl.ds(s * ZROWS, ZROWS)])
        plsc.subcore_barrier()

        def seg_body(g, carry):
            pltpu.sync_copy(srcq_hbm.at[wid, pl.ds(g * SEG, SEG)], srcq_iv)
            pltpu.sync_copy(pidx_hbm.at[wid, pl.ds(g * SEG, SEG)], pidx_iv)
            pltpu.sync_copy(dstr_hbm.at[wid, pl.ds(g * SEG, SEG)], dstr_iv)
            for b in range(DEPTH):
                pltpu.async_copy(xp_hbm.at[srcq_iv.at[b]], bufs[b], gsems[b])

            def body(r, carry2):
                for b in range(DEPTH):
                    k = r * DEPTH + b
                    pltpu.make_async_copy(xp_hbm.at[srcq_iv.at[k]], bufs[b],
                                          gsems[b]).wait()

                    def compact(t, cc):
                        pvec = pidx_iv[k, pl.ds(t * 16, 16)]
                        for l in range(16):
                            i = t * 16 + l
                            off = pvec[l] * H
                            for j in range(8):
                                cbuf[i, pl.ds(j * 16, 16)] = (
                                    bufs[b][i, pl.ds(off + j * 16, 16)])
                        return cc

                    lax.fori_loop(0, CH // 16, compact, 0)
                    pltpu.sync_copy(cbuf, acc_sh.at[dstr_iv.at[k]], add=True)
                    @pl.when(k + DEPTH < SEG)
                    def _():
                        pltpu.async_copy(xp_hbm.at[srcq_iv.at[k + DEPTH]],
                                         bufs[b], gsems[b])
                return carry2

            lax.fori_loop(0, SEG // DEPTH, body, 0)
            return carry

        lax.fori_loop(0, NSEG, seg_body, 0)
        plsc.subcore_barrier()

        pltpu.sync_copy(acc_sh.at[pl.ds(s * ZROWS, ZROWS)],
                        out_hbm.at[c, pl.ds(s * ZROWS, ZROWS)])

    return agg_kernel(xp, srcq, pidx, dstr, zeros)


def _tc_body(x_ref, agg_ref, w1_ref, b1_ref, w2_ref, b2_ref, g_ref, bt_ref,
             out_ref):
    h = x_ref[...] + agg_ref[0, :N] + agg_ref[1, :N]
    h = jnp.dot(h, w1_ref[...], preferred_element_type=jnp.float32)
    h = jnp.maximum(h + b1_ref[...], 0.0)
    h = jnp.dot(h, w2_ref[...], preferred_element_type=jnp.float32)
    h = h + b2_ref[...]
    mean = jnp.mean(h, axis=0, keepdims=True)
    var = jnp.mean((h - mean) * (h - mean), axis=0, keepdims=True)
    out_ref[...] = (h - mean) * lax.rsqrt(var + 1e-5) * g_ref[...] + bt_ref[...]


def kernel(x, edge_index, W1, b1, W2, b2, gamma, beta):
    src = edge_index[0].astype(jnp.int32)
    dst = edge_index[1].astype(jnp.int32)
    pad = E_PAD - E
    tile_of = (jnp.arange(E_PAD, dtype=jnp.int32) // (NCHUNK * CH)) % NS
    dump = N + tile_of
    src_p = jnp.concatenate([src, jnp.zeros((pad,), jnp.int32)])
    dst_p = jnp.concatenate([dst, jnp.zeros((pad,), jnp.int32)])
    valid = jnp.arange(E_PAD, dtype=jnp.int32) < E
    srcq = (src_p // 2).reshape(NW, NCHUNK, CH)
    pidx = (src_p % 2).reshape(NW, NCHUNK, CH)
    dstr = jnp.where(valid, dst_p, dump).reshape(NW, NCHUNK, CH)
    zeros = jnp.zeros((ZROWS, H), jnp.float32)
    xp = x.reshape(N // 2, 2 * H)

    agg = _sc_aggregate(xp, srcq, pidx, dstr, zeros)

    out = pl.pallas_call(
        _tc_body,
        out_shape=jax.ShapeDtypeStruct((N, H), jnp.float32),
    )(x, agg, W1, b1.reshape(1, H), W2, b2.reshape(1, H),
      gamma.reshape(1, H), beta.reshape(1, H))
    return out


# speedup vs baseline: 1.2015x; 1.2015x over previous
import functools

import jax
import jax.numpy as jnp
from jax import lax
from jax.experimental import pallas as pl
from jax.experimental.pallas import tpu as pltpu
from jax.experimental.pallas import tpu_sc as plsc

N = 10000
E = 320000
H = 128

NC = 2
NS = 16
NW = NC * NS
CH = 128
NCHUNK = 80
E_PAD = NW * NCHUNK * CH
DEPTH = 2
SEG = 40
NSEG = NCHUNK // SEG

N_ACC = 10112
ZROWS = N_ACC // NS


def _sc_aggregate(x, srcs, dsts, zeros):
    mesh = plsc.VectorSubcoreMesh(core_axis_name="c", subcore_axis_name="s")

    @functools.partial(
        pl.kernel,
        out_type=jax.ShapeDtypeStruct((NC, N_ACC, H), jnp.float32),
        mesh=mesh,
        scratch_types=[
            pltpu.VMEM((SEG, CH), jnp.int32),
            pltpu.VMEM((SEG, CH), jnp.int32),
            [pltpu.VMEM((CH, H), jnp.float32) for _ in range(DEPTH)],
            pltpu.VMEM_SHARED((N_ACC, H), jnp.float32),
            [pltpu.SemaphoreType.DMA for _ in range(DEPTH)],
        ],
    )
    def agg_kernel(x_hbm, srcs_hbm, dsts_hbm, zeros_hbm, out_hbm,
                   src_iv, dst_iv, bufs, acc_sh, gsems):
        c = lax.axis_index("c")
        s = lax.axis_index("s")
        wid = c * NS + s

        pltpu.sync_copy(zeros_hbm, acc_sh.at[pl.ds(s * ZROWS, ZROWS)])
        plsc.subcore_barrier()

        def seg_body(g, carry):
            pltpu.sync_copy(srcs_hbm.at[wid, pl.ds(g * SEG, SEG)], src_iv)
            pltpu.sync_copy(dsts_hbm.at[wid, pl.ds(g * SEG, SEG)], dst_iv)
            for b in range(DEPTH):
                pltpu.async_copy(x_hbm.at[src_iv.at[b]], bufs[b], gsems[b])

            def body(r, carry2):
                for b in range(DEPTH):
                    k = r * DEPTH + b
                    pltpu.make_async_copy(x_hbm.at[src_iv.at[k]], bufs[b],
                                          gsems[b]).wait()
                    pltpu.sync_copy(bufs[b], acc_sh.at[dst_iv.at[k]],
                                    add=True)
                    @pl.when(k + DEPTH < SEG)
                    def _():
                        pltpu.async_copy(x_hbm.at[src_iv.at[k + DEPTH]],
                                         bufs[b], gsems[b])
                return carry2

            lax.fori_loop(0, SEG // DEPTH, body, 0)
            return carry

        lax.fori_loop(0, NSEG, seg_body, 0)
        plsc.subcore_barrier()

        pltpu.sync_copy(acc_sh.at[pl.ds(s * ZROWS, ZROWS)],
                        out_hbm.at[c, pl.ds(s * ZROWS, ZROWS)])

    return agg_kernel(x, srcs, dsts, zeros)


def _tc_body(x_ref, agg_ref, w1_ref, b1_ref, w2_ref, b2_ref, g_ref, bt_ref,
             out_ref):
    h = x_ref[...] + agg_ref[0, :N] + agg_ref[1, :N]
    h = jnp.dot(h, w1_ref[...], preferred_element_type=jnp.float32)
    h = jnp.maximum(h + b1_ref[...], 0.0)
    h = jnp.dot(h, w2_ref[...], preferred_element_type=jnp.float32)
    h = h + b2_ref[...]
    mean = jnp.mean(h, axis=0, keepdims=True)
    var = jnp.mean((h - mean) * (h - mean), axis=0, keepdims=True)
    out_ref[...] = (h - mean) * lax.rsqrt(var + 1e-5) * g_ref[...] + bt_ref[...]


def kernel(x, edge_index, W1, b1, W2, b2, gamma, beta):
    src = edge_index[0].astype(jnp.int32)
    dst = edge_index[1].astype(jnp.int32)
    pad = E_PAD - E
    srcs = jnp.concatenate([src, jnp.zeros((pad,), jnp.int32)])
    dsts = jnp.concatenate([dst, jnp.full((pad,), N, jnp.int32)])
    srcs = srcs.reshape(NW, NCHUNK, CH)
    dsts = dsts.reshape(NW, NCHUNK, CH)
    zeros = jnp.zeros((ZROWS, H), jnp.float32)

    agg = _sc_aggregate(x, srcs, dsts, zeros)

    out = pl.pallas_call(
        _tc_body,
        out_shape=jax.ShapeDtypeStruct((N, H), jnp.float32),
    )(x, agg, W1, b1.reshape(1, H), W2, b2.reshape(1, H),
      gamma.reshape(1, H), beta.reshape(1, H))
    return out
